# baseline (speedup 1.0000x reference)
import functools

import jax
import jax.numpy as jnp
from jax import lax
from jax.experimental import pallas as pl
from jax.experimental.pallas import tpu as pltpu

_B, _T, _DA = 16, 512, 768
_KW, _DT, _VOCAB = 8, 512, 8192
_H, _DH, _FF = 12, 64, 3072
_EPS = 1e-5
_SQ = 16
_NQ = 1 + _KW
_R = _H * _SQ
_BB = 4
_NC = _B // _BB

_bf16 = jnp.bfloat16
_f32 = jnp.float32

_CN = (((1,), (1,)), ((), ()))
_CM = (((1,), (0,)), ((), ()))


def _ln(x, g, b):
    m = jnp.mean(x, axis=-1, keepdims=True)
    v = jnp.mean((x - m) ** 2, axis=-1, keepdims=True)
    return (x - m) / jnp.sqrt(v + _EPS) * g + b


def _head_mask(shape, row_axis, col_axis):
    return (lax.broadcasted_iota(jnp.int32, shape, col_axis) // _DH
            == lax.broadcasted_iota(jnp.int32, shape, row_axis))


_VC = 1024
_NVC = _VOCAB // _VC


def _main_kernel(a_ref, cls_ref, wq_ref, bq_ref, wk_ref,
                 wv_hbm, wo_hbm, w1_hbm, w2_hbm,
                 bv_ref, bo_ref, g1_ref, be1_ref, b1_ref, b2_ref,
                 g2_ref, be2_ref, pjw_ref, pjb_ref, ppw_ref, ppb_ref,
                 bng_ref, bnb_ref, te_ref,
                 p_out_ref, kws_out_ref,
                 qzk_s, s1_s, c_s, wv_s, wo_s, w1_s, w2_s,
                 kwr_s, kn_s, den_s, acc_s,
                 sem_v, sem_o, sem_1, sem_2):
    i = pl.program_id(0)

    @pl.when(i == 0)
    def _init():
        pltpu.make_async_copy(wv_hbm, wv_s, sem_v).start()
        pltpu.make_async_copy(wo_hbm, wo_s, sem_o).start()
        pltpu.make_async_copy(w1_hbm, w1_s, sem_1).start()
        pltpu.make_async_copy(w2_hbm, w2_s, sem_2).start()
        cls = cls_ref[...]
        q = (jnp.dot(cls, wq_ref[...], preferred_element_type=_f32)
             + bq_ref[...]) * (1.0 / 8.0)
        hm = _head_mask((_H, 1, _DA), 0, 2)
        qz = jnp.where(hm, jnp.broadcast_to(q[None], (_H, _SQ, _DA)), 0.0)
        qz = qz.reshape(_R, _DA).astype(_bf16)
        qzk = lax.dot_general(qz, wk_ref[...].astype(_bf16), _CN,
                              preferred_element_type=_f32)
        qzk_s[...] = qzk.astype(_bf16)
        s1_s[...] = lax.dot_general(qzk_s[...], cls.astype(_bf16), _CN,
                                    preferred_element_type=_f32)

    @pl.when(i < _NC)
    def _ctx():
        qzk = qzk_s[...]
        s1 = s1_s[...][:, : _NQ]
        m1 = jnp.max(s1, -1, keepdims=True)
        clsx = cls_ref[: _NQ].astype(_bf16)
        for j in range(_BB):
            xa = a_ref[j].astype(_bf16)
            s2 = lax.dot_general(qzk, xa, _CN, preferred_element_type=_f32)
            m = jnp.maximum(m1, jnp.max(s2, -1, keepdims=True))
            e1 = jnp.exp(s1 - m)
            e2 = jnp.exp(s2 - m)
            den = (jnp.sum(e1, -1, keepdims=True)
                   + jnp.sum(e2, -1, keepdims=True))
            c = (lax.dot_general(e1.astype(_bf16), clsx, _CM,
                                 preferred_element_type=_f32)
                 + lax.dot_general(e2.astype(_bf16), xa, _CM,
                                   preferred_element_type=_f32)) / den
            b = i * _BB + j
            c_s[pl.ds(b * _R, _R), :] = c.astype(_bf16)

    @pl.when(i == _NC)
    def _tail():
        pltpu.make_async_copy(wv_hbm, wv_s, sem_v).wait()
        pltpu.make_async_copy(wo_hbm, wo_s, sem_o).wait()
        pltpu.make_async_copy(w1_hbm, w1_s, sem_1).wait()
        pltpu.make_async_copy(w2_hbm, w2_s, sem_2).wait()
        wvb = wv_s[...].astype(_bf16)
        hm4 = _head_mask((1, _H, 1, _DA), 1, 3)
        halves = []
        hb = _B // 2
        for k in range(2):
            c2 = c_s[pl.ds(k * hb * _R, hb * _R), :]
            cw = lax.dot_general(c2, wvb, _CM,
                                 preferred_element_type=_f32)
            halves.append(jnp.sum(
                jnp.where(hm4, cw.reshape(hb, _H, _SQ, _DA), 0.0), axis=1))
        o = jnp.concatenate(halves, axis=0)
        o2 = o.reshape(_B * _SQ, _DA) + bv_ref[...]
        cls256 = jnp.broadcast_to(cls_ref[None], (_B, _SQ, _DA)).reshape(
            _B * _SQ, _DA)
        x1 = cls256 + jnp.dot(o2.astype(_bf16), wo_s[...].astype(_bf16),
                              preferred_element_type=_f32) + bo_ref[...]
        xn = _ln(x1, g1_ref[...], be1_ref[...])
        h = jax.nn.gelu(jnp.dot(xn.astype(_bf16), w1_s[...].astype(_bf16),
                                preferred_element_type=_f32) + b1_ref[...])
        x2 = xn + jnp.dot(h.astype(_bf16), w2_s[...].astype(_bf16),
                          preferred_element_type=_f32) + b2_ref[...]
        xo = _ln(x2, g2_ref[...], be2_ref[...])
        xob = xo.astype(_bf16)
        yp = jnp.dot(xob, ppw_ref[...].astype(_bf16),
                     preferred_element_type=_f32) + ppb_ref[...]
        ykw = jnp.dot(xob, pjw_ref[...].astype(_bf16),
                      preferred_element_type=_f32) + pjb_ref[...]
        p_out_ref[...] = yp.reshape(_B, _SQ, _DT)[:, 0:1, :]
        kwr_s[...] = ykw.reshape(_B, _SQ, _DT)[:, 1:_NQ, :]

    @pl.when(i == _NC + 1)
    def _bn():
        kw = kwr_s[...]
        mu = jnp.mean(kw, axis=0, keepdims=True)
        var = jnp.mean((kw - mu) ** 2, axis=0, keepdims=True)
        kwn = (kw - mu) / jnp.sqrt(var + _EPS) * bng_ref[...] + bnb_ref[...]
        kn = kwn / (jnp.sqrt(jnp.sum(kwn * kwn, -1, keepdims=True)) + 1e-8)
        kn_s[...] = kn.reshape(_B * _KW, _DT).astype(_bf16)
        den_s[...] = jnp.zeros((_B * _KW, 1), _f32)
        acc_s[...] = jnp.zeros((_B * _KW, _DT), _f32)

    @pl.when(i > _NC)
    def _vq():
        te_c = te_ref[...]
        teb = te_c.astype(_bf16)
        tinv = 1.0 / (jnp.sqrt(jnp.sum(te_c * te_c, -1, keepdims=True))
                      + 1e-8)
        cos = lax.dot_general(kn_s[...], teb, _CN,
                              preferred_element_type=_f32) * tinv.reshape(
                                  1, _VC)
        e = jnp.exp(cos)
        den_s[...] = den_s[...] + jnp.sum(e, -1, keepdims=True)
        acc_s[...] = acc_s[...] + lax.dot_general(
            e.astype(_bf16), teb, _CM, preferred_element_type=_f32)

    @pl.when(i == _NC + _NVC)
    def _fin():
        kws_out_ref[...] = (acc_s[...] / den_s[...]).reshape(_B, _KW, _DT)


def _const(shape):
    nd = len(shape)
    return pl.BlockSpec(shape, lambda b: (0,) * nd)


@functools.partial(jax.jit)
def kernel(audio_feat, params, token_emb):
    p = params
    cls9 = jnp.concatenate([p['parallel_cls'][0], p['cascaded_cls'][0]], axis=0)
    cls16 = jnp.pad(cls9, ((0, _SQ - _NQ), (0, 0)))
    row = lambda a: a.reshape(1, -1)
    hbm = pl.BlockSpec(memory_space=pltpu.MemorySpace.HBM)

    p_out, keywords = pl.pallas_call(
        _main_kernel,
        grid=(_NC + 1 + _NVC,),
        in_specs=[
            pl.BlockSpec((_BB, _T, _DA),
                         lambda i: (jnp.minimum(i, _NC - 1), 0, 0)),
            _const((_SQ, _DA)),
            _const((_DA, _DA)), _const((1, _DA)),
            _const((_DA, _DA)),
            hbm, hbm, hbm, hbm,
            _const((1, _DA)), _const((1, _DA)),
            _const((1, _DA)), _const((1, _DA)),
            _const((1, _FF)), _const((1, _DA)),
            _const((1, _DA)), _const((1, _DA)),
            _const((_DA, _DT)), _const((1, _DT)),
            _const((_DA, _DT)), _const((1, _DT)),
            _const((1, 1, _DT)), _const((1, 1, _DT)),
            pl.BlockSpec((_VC, _DT),
                         lambda i: (jnp.clip(i - _NC - 1, 0, _NVC - 1), 0)),
        ],
        out_specs=[_const((_B, 1, _DT)), _const((_B, _KW, _DT))],
        out_shape=[jax.ShapeDtypeStruct((_B, 1, _DT), _f32),
                   jax.ShapeDtypeStruct((_B, _KW, _DT), _f32)],
        scratch_shapes=[
            pltpu.VMEM((_R, _DA), _bf16),
            pltpu.VMEM((_R, _SQ), _f32),
            pltpu.VMEM((_B * _R, _DA), _bf16),
            pltpu.VMEM((_DA, _DA), _f32),
            pltpu.VMEM((_DA, _DA), _f32),
            pltpu.VMEM((_DA, _FF), _f32),
            pltpu.VMEM((_FF, _DA), _f32),
            pltpu.VMEM((_B, _KW, _DT), _f32),
            pltpu.VMEM((_B * _KW, _DT), _bf16),
            pltpu.VMEM((_B * _KW, 1), _f32),
            pltpu.VMEM((_B * _KW, _DT), _f32),
            pltpu.SemaphoreType.DMA,
            pltpu.SemaphoreType.DMA,
            pltpu.SemaphoreType.DMA,
            pltpu.SemaphoreType.DMA,
        ],
        compiler_params=pltpu.CompilerParams(
            dimension_semantics=("arbitrary",)),
    )(audio_feat, cls16, p['Wq'], row(p['bq']), p['Wk'],
      p['Wv'], p['Wo'], p['ffn_W1'], p['ffn_W2'],
      row(p['bv']), row(p['bo']),
      row(p['ln1_g']), row(p['ln1_b']),
      row(p['ffn_b1']), row(p['ffn_b2']),
      row(p['ln2_g']), row(p['ln2_b']),
      p['proj_W'], row(p['proj_b']), p['pproj_W'], row(p['pproj_b']),
      p['bn_g'].reshape(1, 1, _DT), p['bn_b'].reshape(1, 1, _DT),
      token_emb)

    return jnp.concatenate([p_out, keywords], axis=1)


# speedup vs baseline: 1.0030x; 1.0030x over previous
import functools

import jax
import jax.numpy as jnp
from jax import lax
from jax.experimental import pallas as pl
from jax.experimental.pallas import tpu as pltpu

_B, _T, _DA = 16, 512, 768
_KW, _DT, _VOCAB = 8, 512, 8192
_H, _DH, _FF = 12, 64, 3072
_EPS = 1e-5
_SQ = 16
_NQ = 1 + _KW
_R = _H * _SQ
_BB = 2
_NC = _B // _BB

_bf16 = jnp.bfloat16
_f32 = jnp.float32

_CN = (((1,), (1,)), ((), ()))
_CM = (((1,), (0,)), ((), ()))


def _ln(x, g, b):
    m = jnp.mean(x, axis=-1, keepdims=True)
    v = jnp.mean((x - m) ** 2, axis=-1, keepdims=True)
    return (x - m) / jnp.sqrt(v + _EPS) * g + b


def _head_mask(shape, row_axis, col_axis):
    return (lax.broadcasted_iota(jnp.int32, shape, col_axis) // _DH
            == lax.broadcasted_iota(jnp.int32, shape, row_axis))


_VC = 2048
_NVC = _VOCAB // _VC


def _main_kernel(a_ref, cls_ref, wq_ref, bq_ref, wk_ref,
                 wv_hbm, wo_hbm, w1_hbm, w2_hbm,
                 bv_ref, bo_ref, g1_ref, be1_ref, b1_ref, b2_ref,
                 g2_ref, be2_ref, pjw_hbm, pjb_ref, ppw_hbm, ppb_ref,
                 bng_ref, bnb_ref, te_ref,
                 p_out_ref, kws_out_ref,
                 qzk_s, s1_s, c_s, wv_s, wo_s, w1_s, w2_s,
                 pjw_s, ppw_s,
                 kwr_s, kn_s, den_s, acc_s,
                 sem_v, sem_o, sem_1, sem_2, sem_pj, sem_pp):
    i = pl.program_id(0)

    @pl.when(i == 0)
    def _init():
        pltpu.make_async_copy(wv_hbm, wv_s, sem_v).start()
        pltpu.make_async_copy(wo_hbm, wo_s, sem_o).start()
        pltpu.make_async_copy(w1_hbm, w1_s, sem_1).start()
        pltpu.make_async_copy(w2_hbm, w2_s, sem_2).start()
        pltpu.make_async_copy(pjw_hbm, pjw_s, sem_pj).start()
        pltpu.make_async_copy(ppw_hbm, ppw_s, sem_pp).start()
        cls = cls_ref[...]
        q = (jnp.dot(cls, wq_ref[...], preferred_element_type=_f32)
             + bq_ref[...]) * (1.0 / 8.0)
        hm = _head_mask((_H, 1, _DA), 0, 2)
        qz = jnp.where(hm, jnp.broadcast_to(q[None], (_H, _SQ, _DA)), 0.0)
        qz = qz.reshape(_R, _DA).astype(_bf16)
        qzk = lax.dot_general(qz, wk_ref[...].astype(_bf16), _CN,
                              preferred_element_type=_f32)
        qzk_s[...] = qzk.astype(_bf16)
        s1_s[...] = lax.dot_general(qzk_s[...], cls.astype(_bf16), _CN,
                                    preferred_element_type=_f32)

    @pl.when(i < _NC)
    def _ctx():
        qzk = qzk_s[...]
        s1 = s1_s[...][:, : _NQ]
        m1 = jnp.max(s1, -1, keepdims=True)
        clsx = cls_ref[: _NQ].astype(_bf16)
        for j in range(_BB):
            xa = a_ref[j].astype(_bf16)
            s2 = lax.dot_general(qzk, xa, _CN, preferred_element_type=_f32)
            m = jnp.maximum(m1, jnp.max(s2, -1, keepdims=True))
            e1 = jnp.exp(s1 - m)
            e2 = jnp.exp(s2 - m)
            den = (jnp.sum(e1, -1, keepdims=True)
                   + jnp.sum(e2, -1, keepdims=True))
            c = (lax.dot_general(e1.astype(_bf16), clsx, _CM,
                                 preferred_element_type=_f32)
                 + lax.dot_general(e2.astype(_bf16), xa, _CM,
                                   preferred_element_type=_f32)) / den
            b = i * _BB + j
            c_s[pl.ds(b * _R, _R), :] = c.astype(_bf16)

    @pl.when(i == _NC)
    def _tail():
        pltpu.make_async_copy(wv_hbm, wv_s, sem_v).wait()
        pltpu.make_async_copy(wo_hbm, wo_s, sem_o).wait()
        pltpu.make_async_copy(w1_hbm, w1_s, sem_1).wait()
        pltpu.make_async_copy(w2_hbm, w2_s, sem_2).wait()
        pltpu.make_async_copy(pjw_hbm, pjw_s, sem_pj).wait()
        pltpu.make_async_copy(ppw_hbm, ppw_s, sem_pp).wait()
        wvb = wv_s[...].astype(_bf16)
        hm4 = _head_mask((1, _H, 1, _DA), 1, 3)
        halves = []
        hb = _B // 2
        for k in range(2):
            c2 = c_s[pl.ds(k * hb * _R, hb * _R), :]
            cw = lax.dot_general(c2, wvb, _CM,
                                 preferred_element_type=_f32)
            halves.append(jnp.sum(
                jnp.where(hm4, cw.reshape(hb, _H, _SQ, _DA), 0.0), axis=1))
        o = jnp.concatenate(halves, axis=0)
        o2 = o.reshape(_B * _SQ, _DA) + bv_ref[...]
        cls256 = jnp.broadcast_to(cls_ref[None], (_B, _SQ, _DA)).reshape(
            _B * _SQ, _DA)
        x1 = cls256 + jnp.dot(o2.astype(_bf16), wo_s[...].astype(_bf16),
                              preferred_element_type=_f32) + bo_ref[...]
        xn = _ln(x1, g1_ref[...], be1_ref[...])
        h = jax.nn.gelu(jnp.dot(xn.astype(_bf16), w1_s[...].astype(_bf16),
                                preferred_element_type=_f32) + b1_ref[...])
        x2 = xn + jnp.dot(h.astype(_bf16), w2_s[...].astype(_bf16),
                          preferred_element_type=_f32) + b2_ref[...]
        xo = _ln(x2, g2_ref[...], be2_ref[...])
        xob = xo.astype(_bf16)
        yp = jnp.dot(xob, ppw_s[...].astype(_bf16),
                     preferred_element_type=_f32) + ppb_ref[...]
        ykw = jnp.dot(xob, pjw_s[...].astype(_bf16),
                      preferred_element_type=_f32) + pjb_ref[...]
        p_out_ref[...] = yp.reshape(_B, _SQ, _DT)[:, 0:1, :]
        kwr_s[...] = ykw.reshape(_B, _SQ, _DT)[:, 1:_NQ, :]

    @pl.when(i == _NC + 1)
    def _bn():
        kw = kwr_s[...]
        mu = jnp.mean(kw, axis=0, keepdims=True)
        var = jnp.mean((kw - mu) ** 2, axis=0, keepdims=True)
        kwn = (kw - mu) / jnp.sqrt(var + _EPS) * bng_ref[...] + bnb_ref[...]
        kn = kwn / (jnp.sqrt(jnp.sum(kwn * kwn, -1, keepdims=True)) + 1e-8)
        kn_s[...] = kn.reshape(_B * _KW, _DT).astype(_bf16)
        den_s[...] = jnp.zeros((_B * _KW, 1), _f32)
        acc_s[...] = jnp.zeros((_B * _KW, _DT), _f32)

    @pl.when(i > _NC)
    def _vq():
        te_c = te_ref[...]
        teb = te_c.astype(_bf16)
        tinv = 1.0 / (jnp.sqrt(jnp.sum(te_c * te_c, -1, keepdims=True))
                      + 1e-8)
        cos = lax.dot_general(kn_s[...], teb, _CN,
                              preferred_element_type=_f32) * tinv.reshape(
                                  1, _VC)
        e = jnp.exp(cos)
        den_s[...] = den_s[...] + jnp.sum(e, -1, keepdims=True)
        acc_s[...] = acc_s[...] + lax.dot_general(
            e.astype(_bf16), teb, _CM, preferred_element_type=_f32)

    @pl.when(i == _NC + _NVC)
    def _fin():
        kws_out_ref[...] = (acc_s[...] / den_s[...]).reshape(_B, _KW, _DT)


def _const(shape):
    nd = len(shape)
    return pl.BlockSpec(shape, lambda b: (0,) * nd)


@functools.partial(jax.jit)
def kernel(audio_feat, params, token_emb):
    p = params
    cls9 = jnp.concatenate([p['parallel_cls'][0], p['cascaded_cls'][0]], axis=0)
    cls16 = jnp.pad(cls9, ((0, _SQ - _NQ), (0, 0)))
    row = lambda a: a.reshape(1, -1)
    hbm = pl.BlockSpec(memory_space=pltpu.MemorySpace.HBM)

    p_out, keywords = pl.pallas_call(
        _main_kernel,
        grid=(_NC + 1 + _NVC,),
        in_specs=[
            pl.BlockSpec((_BB, _T, _DA),
                         lambda i: (jnp.minimum(i, _NC - 1), 0, 0)),
            _const((_SQ, _DA)),
            _const((_DA, _DA)), _const((1, _DA)),
            _const((_DA, _DA)),
            hbm, hbm, hbm, hbm,
            _const((1, _DA)), _const((1, _DA)),
            _const((1, _DA)), _const((1, _DA)),
            _const((1, _FF)), _const((1, _DA)),
            _const((1, _DA)), _const((1, _DA)),
            hbm, _const((1, _DT)),
            hbm, _const((1, _DT)),
            _const((1, 1, _DT)), _const((1, 1, _DT)),
            pl.BlockSpec((_VC, _DT),
                         lambda i: (jnp.clip(i - _NC - 1, 0, _NVC - 1), 0)),
        ],
        out_specs=[_const((_B, 1, _DT)), _const((_B, _KW, _DT))],
        out_shape=[jax.ShapeDtypeStruct((_B, 1, _DT), _f32),
                   jax.ShapeDtypeStruct((_B, _KW, _DT), _f32)],
        scratch_shapes=[
            pltpu.VMEM((_R, _DA), _bf16),
            pltpu.VMEM((_R, _SQ), _f32),
            pltpu.VMEM((_B * _R, _DA), _bf16),
            pltpu.VMEM((_DA, _DA), _f32),
            pltpu.VMEM((_DA, _DA), _f32),
            pltpu.VMEM((_DA, _FF), _f32),
            pltpu.VMEM((_FF, _DA), _f32),
            pltpu.VMEM((_DA, _DT), _f32),
            pltpu.VMEM((_DA, _DT), _f32),
            pltpu.VMEM((_B, _KW, _DT), _f32),
            pltpu.VMEM((_B * _KW, _DT), _bf16),
            pltpu.VMEM((_B * _KW, 1), _f32),
            pltpu.VMEM((_B * _KW, _DT), _f32),
            pltpu.SemaphoreType.DMA,
            pltpu.SemaphoreType.DMA,
            pltpu.SemaphoreType.DMA,
            pltpu.SemaphoreType.DMA,
            pltpu.SemaphoreType.DMA,
            pltpu.SemaphoreType.DMA,
        ],
        compiler_params=pltpu.CompilerParams(
            dimension_semantics=("arbitrary",)),
    )(audio_feat, cls16, p['Wq'], row(p['bq']), p['Wk'],
      p['Wv'], p['Wo'], p['ffn_W1'], p['ffn_W2'],
      row(p['bv']), row(p['bo']),
      row(p['ln1_g']), row(p['ln1_b']),
      row(p['ffn_b1']), row(p['ffn_b2']),
      row(p['ln2_g']), row(p['ln2_b']),
      p['proj_W'], row(p['proj_b']), p['pproj_W'], row(p['pproj_b']),
      p['bn_g'].reshape(1, 1, _DT), p['bn_b'].reshape(1, 1, _DT),
      token_emb)

    return jnp.concatenate([p_out, keywords], axis=1)
